# baseline (speedup 1.0000x reference)
import math

import jax
import jax.numpy as jnp
from jax.experimental import pallas as pl
from jax.experimental.pallas import tpu as pltpu

_NCH = 8


def _gen_body(p_hbm, o_hbm, tab, sems):
    n_batch, seq_len, d = o_hbm.shape
    chr_ = seq_len // _NCH
    neg_log2n_scale = -2.0 / d * math.log2(10000.0)

    col = jax.lax.broadcasted_iota(jnp.int32, (chr_, d), 1)
    row0 = jax.lax.broadcasted_iota(jnp.int32, (chr_, d), 0)
    inv = jnp.exp2((col // 2).astype(jnp.float32) * neg_log2n_scale)
    phase = jnp.where(col % 2 == 0, 0.0, math.pi / 2).astype(jnp.float32)

    for ch in range(_NCH):
        rows = (row0 + ch * chr_).astype(jnp.float32)
        tab[pl.ds(ch * chr_, chr_), :] = jnp.sin(rows * inv + phase)
        for b in range(n_batch):
            pltpu.make_async_copy(
                tab.at[pl.ds(ch * chr_, chr_)],
                o_hbm.at[b, pl.ds(ch * chr_, chr_)],
                sems.at[ch]).start()
    for ch in range(_NCH):
        for b in range(n_batch):
            pltpu.make_async_copy(
                tab.at[pl.ds(ch * chr_, chr_)],
                o_hbm.at[b, pl.ds(ch * chr_, chr_)],
                sems.at[ch]).wait()


def kernel(inputs, P):
    b, s = inputs.shape
    d = P.shape[1]
    return pl.pallas_call(
        _gen_body,
        in_specs=[pl.BlockSpec(memory_space=pltpu.MemorySpace.HBM)],
        out_specs=pl.BlockSpec(memory_space=pltpu.MemorySpace.HBM),
        out_shape=jax.ShapeDtypeStruct((b, s, d), P.dtype),
        scratch_shapes=[
            pltpu.VMEM((s, d), P.dtype),
            pltpu.SemaphoreType.DMA((_NCH,)),
        ],
    )(P)


# speedup vs baseline: 2.5531x; 2.5531x over previous
import jax
import jax.numpy as jnp
from jax.experimental import pallas as pl
from jax.experimental.pallas import tpu as pltpu

_NCH = 8


def _dma_body(p_hbm, o_hbm, vbuf, in_sems, out_sems):
    n_batch = o_hbm.shape[0]
    ch_rows = p_hbm.shape[0] // _NCH

    def in_copy(ch):
        return pltpu.make_async_copy(
            p_hbm.at[pl.ds(ch * ch_rows, ch_rows)],
            vbuf.at[pl.ds(ch * ch_rows, ch_rows)],
            in_sems.at[ch])

    def out_copy(ch, b):
        return pltpu.make_async_copy(
            vbuf.at[pl.ds(ch * ch_rows, ch_rows)],
            o_hbm.at[b, pl.ds(ch * ch_rows, ch_rows)],
            out_sems.at[ch])

    for ch in range(_NCH):
        in_copy(ch).start()
    for ch in range(_NCH):
        in_copy(ch).wait()
        for b in range(n_batch):
            out_copy(ch, b).start(priority=b % 2)
    for ch in range(_NCH):
        for b in range(n_batch):
            out_copy(ch, b).wait()


def kernel(inputs, P):
    b, s = inputs.shape
    d = P.shape[1]
    return pl.pallas_call(
        _dma_body,
        in_specs=[pl.BlockSpec(memory_space=pltpu.MemorySpace.HBM)],
        out_specs=pl.BlockSpec(memory_space=pltpu.MemorySpace.HBM),
        out_shape=jax.ShapeDtypeStruct((b, s, d), P.dtype),
        scratch_shapes=[
            pltpu.VMEM((s, d), P.dtype),
            pltpu.SemaphoreType.DMA((_NCH,)),
            pltpu.SemaphoreType.DMA((_NCH,)),
        ],
    )(P)
